# baseline (speedup 1.0000x reference)
import functools

import jax
import jax.numpy as jnp
from jax import lax
from jax.experimental import pallas as pl
from jax.experimental.pallas import tpu as pltpu
from jax.experimental.pallas import tpu_sc as plsc

N_NODES = 10000
N_EDGES = 320000
D_FEAT = 128
DH = 64
D2 = 2 * D_FEAT

NC = 2
NS = 16

C = 40
EDGES_PER_SUB = N_EDGES // NS
CHUNKS_PER_SUB = EDGES_PER_SUB // C

DRAIN_ROWS = 40
N_BLOCKS = N_NODES // DRAIN_ROWS
BLOCK_ROUNDS = -(-N_BLOCKS // NS)

_LOG1P_C = (
    0.04106444225260315,
    -0.15602827499078686,
    0.30467224693119505,
    -0.4963682486301464,
    0.9998879230599648,
)


def _node_table_kernel(x_ref, w1_ref, b1_ref, w2_ref, d_ref, s_ref):
    xb = x_ref[...]
    d_ref[0] = (
        jnp.dot(xb, w1_ref[0], preferred_element_type=jnp.float32) + b1_ref[0]
    )
    s_ref[0] = jnp.dot(xb, w2_ref[0], preferred_element_type=jnp.float32)


def _edge_table_kernel(eat_ref, we_ref, e_ref):
    e_ref[0] = lax.dot_general(
        eat_ref[...], we_ref[0],
        dimension_numbers=(((0,), (0,)), ((), ())),
        preferred_element_type=jnp.float32)


def _compute_chunk(d_rows, s_rows, e_rows, msg):
    @plsc.parallel_loop(0, C, 1, unroll=6)
    def edge_body(e):
        for v in range(DH // 16):
            lo = v * 16
            hi = DH + v * 16
            f = (d_rows[e, pl.ds(lo, 16)] + s_rows[e, pl.ds(lo, 16)]
                 + e_rows[e, pl.ds(lo, 16)])
            s = (d_rows[e, pl.ds(hi, 16)] + s_rows[e, pl.ds(hi, 16)]
                 + e_rows[e, pl.ds(hi, 16)])
            gate = 1.0 / (1.0 + jnp.exp(-f))
            u = jnp.exp(-jnp.abs(s))
            q = _LOG1P_C[0]
            for ck in _LOG1P_C[1:]:
                q = q * u + ck
            sp = jnp.maximum(s, 0.0) + u * q
            msg[e, pl.ds(lo, 16)] = gate * sp


def _sc_edge_kernel(d_tab, s_tab, e_all, dst3, src3, xs, out,
                    idx_dst, idx_src,
                    d0, d1, s0, s1, e0, e1, m0, m1, acc,
                    gd0, gs0, ge0, gd1, gs1, ge1, ss0, ss1):
    cid = lax.axis_index("c")
    sid = lax.axis_index("s")

    d_bufs = (d0, d1)
    s_bufs = (s0, s1)
    e_bufs = (e0, e1)
    m_bufs = (m0, m1)
    g_sems = ((gd0, gs0, ge0), (gd1, gs1, ge1))
    s_sems = (ss0, ss1)

    my_d = d_tab.at[cid]
    my_s = s_tab.at[cid]
    my_e = e_all.at[cid]
    my_out = out.at[cid]
    ebase = sid * EDGES_PER_SUB

    pltpu.sync_copy(dst3.at[sid], idx_dst)
    pltpu.sync_copy(src3.at[sid], idx_src)

    my_x = xs.at[cid]
    for j in range(BLOCK_ROUNDS):
        blk = j * NS + sid

        @pl.when(blk < N_BLOCKS)
        def _():
            rows = pl.ds(blk * DRAIN_ROWS, DRAIN_ROWS)
            pltpu.sync_copy(my_x.at[rows], m0)
            pltpu.sync_copy(m0, acc.at[rows])

    plsc.subcore_barrier()

    def issue_gathers(i, b):
        pltpu.async_copy(my_d.at[idx_dst.at[i]], d_bufs[b], g_sems[b][0])
        pltpu.async_copy(my_s.at[idx_src.at[i]], s_bufs[b], g_sems[b][1])
        pltpu.async_copy(my_e.at[pl.ds(ebase + i * C, C)], e_bufs[b],
                         g_sems[b][2])

    issue_gathers(0, 0)
    issue_gathers(1, 1)

    def outer_body(g, carry):
        for b in range(2):
            i = g * 2 + b

            @pl.when(i >= 2)
            def _():
                pltpu.make_async_copy(my_out.at[pl.ds(0, C)], m_bufs[b],
                                      s_sems[b]).wait()

            pltpu.make_async_copy(my_d.at[pl.ds(0, C)], d_bufs[b],
                                  g_sems[b][0]).wait()
            pltpu.make_async_copy(my_s.at[pl.ds(0, C)], s_bufs[b],
                                  g_sems[b][1]).wait()
            pltpu.make_async_copy(my_e.at[pl.ds(0, C)], e_bufs[b],
                                  g_sems[b][2]).wait()

            _compute_chunk(d_bufs[b], s_bufs[b], e_bufs[b], m_bufs[b])

            pltpu.async_copy(m_bufs[b], acc.at[idx_dst.at[i]], s_sems[b],
                             add=True)

            @pl.when(i < CHUNKS_PER_SUB - 2)
            def _():
                pltpu.async_copy(my_d.at[idx_dst.at[i + 2]], d_bufs[b],
                                 g_sems[b][0])
                pltpu.async_copy(my_s.at[idx_src.at[i + 2]], s_bufs[b],
                                 g_sems[b][1])
                pltpu.async_copy(my_e.at[pl.ds(ebase + (i + 2) * C, C)],
                                 e_bufs[b], g_sems[b][2])

        return carry

    lax.fori_loop(0, CHUNKS_PER_SUB // 2, outer_body, 0)

    pltpu.make_async_copy(my_out.at[pl.ds(0, C)], m_bufs[0], s_sems[0]).wait()
    pltpu.make_async_copy(my_out.at[pl.ds(0, C)], m_bufs[1], s_sems[1]).wait()

    plsc.subcore_barrier()

    for j in range(BLOCK_ROUNDS):
        blk = j * NS + sid

        @pl.when(blk < N_BLOCKS)
        def _():
            start = blk * DRAIN_ROWS
            pltpu.sync_copy(acc.at[pl.ds(start, DRAIN_ROWS)], m0)

            @plsc.parallel_loop(0, DRAIN_ROWS * (DH // 16), 1, unroll=4)
            def _relu(i):
                r = i // (DH // 16)
                c = (i % (DH // 16)) * 16
                m0[r, pl.ds(c, 16)] = jnp.maximum(m0[r, pl.ds(c, 16)], 0.0)

            pltpu.sync_copy(m0, my_out.at[pl.ds(start, DRAIN_ROWS)])


def _sc_scratch_types():
    return [
        pltpu.VMEM((CHUNKS_PER_SUB, C), jnp.int32),
        pltpu.VMEM((CHUNKS_PER_SUB, C), jnp.int32),
        pltpu.VMEM((C, D_FEAT), jnp.float32),
        pltpu.VMEM((C, D_FEAT), jnp.float32),
        pltpu.VMEM((C, D_FEAT), jnp.float32),
        pltpu.VMEM((C, D_FEAT), jnp.float32),
        pltpu.VMEM((C, D_FEAT), jnp.float32),
        pltpu.VMEM((C, D_FEAT), jnp.float32),
        pltpu.VMEM((C, DH), jnp.float32),
        pltpu.VMEM((C, DH), jnp.float32),
        pltpu.VMEM_SHARED((N_NODES, DH), jnp.float32),
    ] + [pltpu.SemaphoreType.DMA] * 8


@jax.jit
def kernel(x, edge_index, edge_attr, W_f, b_f, W_s, b_s):
    src = edge_index[0].astype(jnp.int32)
    dst = edge_index[1].astype(jnp.int32)
    dst3 = dst.reshape(NS, CHUNKS_PER_SUB, C)
    src3 = src.reshape(NS, CHUNKS_PER_SUB, C)

    def split_cols(w):
        return jnp.stack([
            jnp.concatenate([w[:, :DH], w[:, D_FEAT:D_FEAT + DH]], axis=1),
            jnp.concatenate([w[:, DH:D_FEAT], w[:, D_FEAT + DH:]], axis=1),
        ])

    Wfs = jnp.concatenate([W_f, W_s], axis=1)
    bfs = jnp.concatenate([b_f, b_s]).reshape(1, D2)
    W1r = split_cols(Wfs[:D_FEAT])
    W2r = split_cols(Wfs[D_FEAT:D2])
    Wer = split_cols(Wfs[D2:])
    b1r = split_cols(bfs)

    bn = 1000
    d_tab, s_tab = pl.pallas_call(
        _node_table_kernel,
        grid=(NC, N_NODES // bn),
        in_specs=[
            pl.BlockSpec((bn, D_FEAT), lambda c, i: (i, 0)),
            pl.BlockSpec((1, D_FEAT, D_FEAT), lambda c, i: (c, 0, 0)),
            pl.BlockSpec((1, 1, D_FEAT), lambda c, i: (c, 0, 0)),
            pl.BlockSpec((1, D_FEAT, D_FEAT), lambda c, i: (c, 0, 0)),
        ],
        out_specs=[
            pl.BlockSpec((1, bn, D_FEAT), lambda c, i: (c, i, 0)),
            pl.BlockSpec((1, bn, D_FEAT), lambda c, i: (c, i, 0)),
        ],
        out_shape=[
            jax.ShapeDtypeStruct((NC, N_NODES, D_FEAT), jnp.float32),
            jax.ShapeDtypeStruct((NC, N_NODES, D_FEAT), jnp.float32),
        ],
    )(x, W1r, b1r, W2r)

    be = 16000
    e_all = pl.pallas_call(
        _edge_table_kernel,
        grid=(NC, N_EDGES // be),
        in_specs=[
            pl.BlockSpec((4, be), lambda c, i: (0, i)),
            pl.BlockSpec((1, 4, D_FEAT), lambda c, i: (c, 0, 0)),
        ],
        out_specs=pl.BlockSpec((1, be, D_FEAT), lambda c, i: (c, i, 0)),
        out_shape=jax.ShapeDtypeStruct((NC, N_EDGES, D_FEAT), jnp.float32),
    )(edge_attr.T, Wer)

    xs = jnp.stack([x[:, :DH], x[:, DH:]])

    sc_call = functools.partial(
        pl.kernel,
        out_type=jax.ShapeDtypeStruct((NC, N_NODES, DH), jnp.float32),
        mesh=plsc.VectorSubcoreMesh(core_axis_name="c", subcore_axis_name="s"),
        scratch_types=_sc_scratch_types(),
        compiler_params=pltpu.CompilerParams(use_tc_tiling_on_sc=False),
    )(_sc_edge_kernel)
    acc2 = sc_call(d_tab, s_tab, e_all, dst3, src3, xs)

    return jnp.concatenate([acc2[0], acc2[1]], axis=1)


# speedup vs baseline: 1.7803x; 1.7803x over previous
import functools

import jax
import jax.numpy as jnp
from jax import lax
from jax.experimental import pallas as pl
from jax.experimental.pallas import tpu as pltpu
from jax.experimental.pallas import tpu_sc as plsc

N_NODES = 10000
N_EDGES = 320000
D_FEAT = 128
DH = 64
D2 = 2 * D_FEAT

NC = 2
NS = 16

C = 40
EDGES_PER_SUB = N_EDGES // NS
CHUNKS_PER_SUB = EDGES_PER_SUB // C

DRAIN_ROWS = 40
N_BLOCKS = N_NODES // DRAIN_ROWS
BLOCK_ROUNDS = -(-N_BLOCKS // NS)

_LOG1P_C = (
    0.04106444225260315,
    -0.15602827499078686,
    0.30467224693119505,
    -0.4963682486301464,
    0.9998879230599648,
)


def _node_table_kernel(x_ref, w1_ref, b1_ref, w2_ref, d_ref, s_ref):
    xb = x_ref[...]
    d_ref[0] = (
        jnp.dot(xb, w1_ref[0], preferred_element_type=jnp.float32) + b1_ref[0]
    )
    s_ref[0] = jnp.dot(xb, w2_ref[0], preferred_element_type=jnp.float32)


def _edge_table_kernel(eat_ref, we_ref, e_ref):
    e_ref[0] = lax.dot_general(
        eat_ref[...], we_ref[0],
        dimension_numbers=(((0,), (0,)), ((), ())),
        preferred_element_type=jnp.float32)


def _compute_chunk(d_rows, s_rows, e_rows, msg):
    @plsc.parallel_loop(0, C, 1, unroll=4)
    def edge_body(e):
        for v in range(DH // 16):
            lo = v * 16
            hi = DH + v * 16
            f = (d_rows[e, pl.ds(lo, 16)] + s_rows[e, pl.ds(lo, 16)]
                 + e_rows[e, pl.ds(lo, 16)])
            s = (d_rows[e, pl.ds(hi, 16)] + s_rows[e, pl.ds(hi, 16)]
                 + e_rows[e, pl.ds(hi, 16)])
            gate = 1.0 / (1.0 + jnp.exp(-f))
            u = jnp.exp(-jnp.abs(s))
            q = _LOG1P_C[0]
            for ck in _LOG1P_C[1:]:
                q = q * u + ck
            sp = jnp.maximum(s, 0.0) + u * q
            msg[e, pl.ds(lo, 16)] = gate * sp


def _sc_edge_kernel(d_tab, s_tab, e_all, dst3, src3, xs, out,
                    idx_dst, idx_src,
                    d0, d1, s0, s1, e0, e1, m0, m1, acc,
                    gd0, gs0, ge0, gd1, gs1, ge1, ss0, ss1):
    cid = lax.axis_index("c")
    sid = lax.axis_index("s")

    d_bufs = (d0, d1)
    s_bufs = (s0, s1)
    e_bufs = (e0, e1)
    m_bufs = (m0, m1)
    g_sems = ((gd0, gs0, ge0), (gd1, gs1, ge1))
    s_sems = (ss0, ss1)

    my_d = d_tab.at[cid]
    my_s = s_tab.at[cid]
    my_e = e_all.at[cid]
    my_out = out.at[cid]
    ebase = sid * EDGES_PER_SUB

    pltpu.sync_copy(dst3.at[sid], idx_dst)
    pltpu.sync_copy(src3.at[sid], idx_src)

    my_x = xs.at[cid]
    for j in range(BLOCK_ROUNDS):
        blk = j * NS + sid

        @pl.when(blk < N_BLOCKS)
        def _():
            rows = pl.ds(blk * DRAIN_ROWS, DRAIN_ROWS)
            pltpu.sync_copy(my_x.at[rows], m0)
            pltpu.sync_copy(m0, acc.at[rows])

    plsc.subcore_barrier()

    def issue_gathers(i, b):
        pltpu.async_copy(my_d.at[idx_dst.at[i]], d_bufs[b], g_sems[b][0])
        pltpu.async_copy(my_s.at[idx_src.at[i]], s_bufs[b], g_sems[b][1])
        pltpu.async_copy(my_e.at[pl.ds(ebase + i * C, C)], e_bufs[b],
                         g_sems[b][2])

    issue_gathers(0, 0)
    issue_gathers(1, 1)

    def outer_body(g, carry):
        for b in range(2):
            i = g * 2 + b

            @pl.when(i >= 2)
            def _():
                pltpu.make_async_copy(my_out.at[pl.ds(0, C)], m_bufs[b],
                                      s_sems[b]).wait()

            pltpu.make_async_copy(my_d.at[pl.ds(0, C)], d_bufs[b],
                                  g_sems[b][0]).wait()
            pltpu.make_async_copy(my_s.at[pl.ds(0, C)], s_bufs[b],
                                  g_sems[b][1]).wait()
            pltpu.make_async_copy(my_e.at[pl.ds(0, C)], e_bufs[b],
                                  g_sems[b][2]).wait()

            _compute_chunk(d_bufs[b], s_bufs[b], e_bufs[b], m_bufs[b])

            pltpu.async_copy(m_bufs[b], acc.at[idx_dst.at[i]], s_sems[b],
                             add=True)

            @pl.when(i < CHUNKS_PER_SUB - 2)
            def _():
                pltpu.async_copy(my_d.at[idx_dst.at[i + 2]], d_bufs[b],
                                 g_sems[b][0])
                pltpu.async_copy(my_s.at[idx_src.at[i + 2]], s_bufs[b],
                                 g_sems[b][1])
                pltpu.async_copy(my_e.at[pl.ds(ebase + (i + 2) * C, C)],
                                 e_bufs[b], g_sems[b][2])

        return carry

    lax.fori_loop(0, CHUNKS_PER_SUB // 2, outer_body, 0)

    pltpu.make_async_copy(my_out.at[pl.ds(0, C)], m_bufs[0], s_sems[0]).wait()
    pltpu.make_async_copy(my_out.at[pl.ds(0, C)], m_bufs[1], s_sems[1]).wait()

    plsc.subcore_barrier()

    for j in range(BLOCK_ROUNDS):
        blk = j * NS + sid

        @pl.when(blk < N_BLOCKS)
        def _():
            start = blk * DRAIN_ROWS
            pltpu.sync_copy(acc.at[pl.ds(start, DRAIN_ROWS)], m0)

            @plsc.parallel_loop(0, DRAIN_ROWS * (DH // 16), 1, unroll=4)
            def _relu(i):
                r = i // (DH // 16)
                c = (i % (DH // 16)) * 16
                m0[r, pl.ds(c, 16)] = jnp.maximum(m0[r, pl.ds(c, 16)], 0.0)

            pltpu.sync_copy(m0, my_out.at[pl.ds(start, DRAIN_ROWS)])


def _sc_scratch_types():
    return [
        pltpu.VMEM((CHUNKS_PER_SUB, C), jnp.int32),
        pltpu.VMEM((CHUNKS_PER_SUB, C), jnp.int32),
        pltpu.VMEM((C, D_FEAT), jnp.float32),
        pltpu.VMEM((C, D_FEAT), jnp.float32),
        pltpu.VMEM((C, D_FEAT), jnp.float32),
        pltpu.VMEM((C, D_FEAT), jnp.float32),
        pltpu.VMEM((C, D_FEAT), jnp.float32),
        pltpu.VMEM((C, D_FEAT), jnp.float32),
        pltpu.VMEM((C, DH), jnp.float32),
        pltpu.VMEM((C, DH), jnp.float32),
        pltpu.VMEM_SHARED((N_NODES, DH), jnp.float32),
    ] + [pltpu.SemaphoreType.DMA] * 8


@jax.jit
def kernel(x, edge_index, edge_attr, W_f, b_f, W_s, b_s):
    src = edge_index[0].astype(jnp.int32)
    dst = edge_index[1].astype(jnp.int32)
    dst3 = dst.reshape(NS, CHUNKS_PER_SUB, C)
    src3 = src.reshape(NS, CHUNKS_PER_SUB, C)

    def split_cols(w):
        return jnp.stack([
            jnp.concatenate([w[:, :DH], w[:, D_FEAT:D_FEAT + DH]], axis=1),
            jnp.concatenate([w[:, DH:D_FEAT], w[:, D_FEAT + DH:]], axis=1),
        ])

    Wfs = jnp.concatenate([W_f, W_s], axis=1)
    bfs = jnp.concatenate([b_f, b_s]).reshape(1, D2)
    W1r = split_cols(Wfs[:D_FEAT])
    W2r = split_cols(Wfs[D_FEAT:D2])
    Wer = split_cols(Wfs[D2:])
    b1r = split_cols(bfs)

    bn = 1000
    d_tab, s_tab = pl.pallas_call(
        _node_table_kernel,
        grid=(NC, N_NODES // bn),
        in_specs=[
            pl.BlockSpec((bn, D_FEAT), lambda c, i: (i, 0)),
            pl.BlockSpec((1, D_FEAT, D_FEAT), lambda c, i: (c, 0, 0)),
            pl.BlockSpec((1, 1, D_FEAT), lambda c, i: (c, 0, 0)),
            pl.BlockSpec((1, D_FEAT, D_FEAT), lambda c, i: (c, 0, 0)),
        ],
        out_specs=[
            pl.BlockSpec((1, bn, D_FEAT), lambda c, i: (c, i, 0)),
            pl.BlockSpec((1, bn, D_FEAT), lambda c, i: (c, i, 0)),
        ],
        out_shape=[
            jax.ShapeDtypeStruct((NC, N_NODES, D_FEAT), jnp.float32),
            jax.ShapeDtypeStruct((NC, N_NODES, D_FEAT), jnp.float32),
        ],
    )(x, W1r, b1r, W2r)

    be = 16000
    e_all = pl.pallas_call(
        _edge_table_kernel,
        grid=(NC, N_EDGES // be),
        in_specs=[
            pl.BlockSpec((4, be), lambda c, i: (0, i)),
            pl.BlockSpec((1, 4, D_FEAT), lambda c, i: (c, 0, 0)),
        ],
        out_specs=pl.BlockSpec((1, be, D_FEAT), lambda c, i: (c, i, 0)),
        out_shape=jax.ShapeDtypeStruct((NC, N_EDGES, D_FEAT), jnp.float32),
    )(edge_attr.T, Wer)

    xs = jnp.stack([x[:, :DH], x[:, DH:]])

    sc_call = functools.partial(
        pl.kernel,
        out_type=jax.ShapeDtypeStruct((NC, N_NODES, DH), jnp.float32),
        mesh=plsc.VectorSubcoreMesh(core_axis_name="c", subcore_axis_name="s"),
        scratch_types=_sc_scratch_types(),
        compiler_params=pltpu.CompilerParams(use_tc_tiling_on_sc=False),
    )(_sc_edge_kernel)
    acc2 = sc_call(d_tab, s_tab, e_all, dst3, src3, xs)

    return jnp.concatenate([acc2[0], acc2[1]], axis=1)
